# Initial kernel scaffold; baseline (speedup 1.0000x reference)
import jax
import jax.numpy as jnp
from jax.experimental import pallas as pl


def kernel(x, edge_index, lin0_W, lin0_b, lin0_g, lin0_beta, conv_Ws, conv_bs, conv_gs, conv_betas, lin1_W, lin1_b, lin1_g, lin1_beta):
    raise NotImplementedError("write your pallas kernel here")


# speedup vs baseline: 1.6653x; 1.6653x over previous
import functools
import jax
import jax.numpy as jnp
from jax import lax
from jax.experimental import pallas as pl

N = 10000
E = 320000
NIN = 128
H = 64
_BN_S = 1.0 / jnp.sqrt(1.0 + 1e-5)
_NEG = -3.4e38


def _mm_kernel(x_ref, w_ref, o_ref):
    o_ref[...] = jnp.dot(x_ref[...], w_ref[...],
                         preferred_element_type=jnp.float32)


def _matmul(x, w, block_rows=2000):
    m, k = x.shape
    k2, n = w.shape
    grid = (m // block_rows,)
    return pl.pallas_call(
        _mm_kernel,
        grid=grid,
        in_specs=[
            pl.BlockSpec((block_rows, k), lambda i: (i, 0)),
            pl.BlockSpec((k2, n), lambda i: (0, 0)),
        ],
        out_specs=pl.BlockSpec((block_rows, n), lambda i: (i, 0)),
        out_shape=jax.ShapeDtypeStruct((m, n), jnp.float32),
    )(x, w)


def kernel(x, edge_index, lin0_W, lin0_b, lin0_g, lin0_beta, conv_Ws,
           conv_bs, conv_gs, conv_betas, lin1_W, lin1_b, lin1_g, lin1_beta):
    src = edge_index[0]
    dst = edge_index[1]

    sg0 = _BN_S * lin0_g
    xn = jax.nn.relu((_matmul(x, lin0_W) + lin0_b) * sg0 + lin0_beta)

    def edge_layer(xn, l):
        W = conv_Ws[l]
        Wt, Wb = W[:H], W[H:]
        sg = _BN_S * conv_gs[l]
        sgn = jnp.where(sg >= 0, 1.0, -1.0)
        Wc = jnp.concatenate([Wt - Wb, Wb * sgn[None, :]], axis=1)
        P = _matmul(xn, Wc)
        A, Beff = P[:, :H], P[:, H:]
        M = jax.ops.segment_max(Beff[src], dst, num_segments=N)
        M = jnp.maximum(M, _NEG)
        c = (A + conv_bs[l]) * sg + conv_betas[l]
        return jax.nn.relu(c + jnp.abs(sg) * M)

    xn = xn - edge_layer(xn, 0)
    xn = edge_layer(xn, 1)
    xn = xn - edge_layer(xn, 2)
    xn = xn - edge_layer(xn, 3)

    sg1 = _BN_S * lin1_g
    out = jax.nn.relu((_matmul(xn, lin1_W) + lin1_b) * sg1 + lin1_beta)
    return jax.nn.log_softmax(out, axis=1)


# speedup vs baseline: 4.2579x; 2.5568x over previous
import functools
import jax
import jax.numpy as jnp
from jax import lax
from jax.experimental import pallas as pl
from jax.experimental.pallas import tpu as pltpu
from jax.experimental.pallas import tpu_sc as plsc

N = 10000
E = 320000
NIN = 128
H = 64
_BN_S = (1.0 + 1e-5) ** -0.5
_NEG = -3.4e38

_NC = 2
_NS = 16
_NW = _NC * _NS
RANGE = 320
NPAD = _NW * RANGE
ACCR = 328
FLUSH = 8192
EPAD = 40 * FLUSH
CHB = 8000
CHE = 512

_sc_mesh = plsc.VectorSubcoreMesh(core_axis_name="c", subcore_axis_name="s",
                                  num_cores=_NC, num_subcores=_NS)


def _worker_id():
    return lax.axis_index("s") * _NC + lax.axis_index("c")


def _bucket_body(src_hbm, dst_hbm, bsrc_hbm, bdst_hbm, cnts_hbm,
                 sbuf, dbuf, osrc, odst, cbuf):
    w = _worker_id()
    lo = w * RANGE
    iota = lax.iota(jnp.int32, 16)

    def init_body(i, _):
        base = i * 16
        osrc[pl.ds(base, 16)] = (base + iota) & 8191
        odst[pl.ds(base, 16)] = jnp.full((16,), RANGE, jnp.int32)
        return 0
    lax.fori_loop(0, (FLUSH + 16) // 16, init_body, 0)

    def chunk_body(ci, carry):
        cb = pl.multiple_of(ci * CHB, 8)
        pltpu.sync_copy(dst_hbm.at[pl.ds(cb, CHB)], dbuf)
        pltpu.sync_copy(src_hbm.at[pl.ds(cb, CHB)], sbuf)

        def vec_body(i, c2):
            off, flushed = c2
            d = dbuf[pl.ds(i * 16, 16)]
            s = sbuf[pl.ds(i * 16, 16)]
            m = (d >= lo) & (d < lo + RANGE)
            cs = plsc.cumsum(jnp.where(m, jnp.int32(1), jnp.int32(0)))
            idx = (off + cs) - 1
            plsc.store_scatter(odst, [idx], d - lo, mask=m)
            plsc.store_scatter(osrc, [idx], s, mask=m)
            off = off + jnp.max(cs)

            def flush(c3):
                off, flushed = c3
                ob = pl.multiple_of(w * EPAD + flushed, FLUSH)
                pltpu.sync_copy(osrc.at[pl.ds(0, FLUSH)],
                                bsrc_hbm.at[pl.ds(ob, FLUSH)])
                pltpu.sync_copy(odst.at[pl.ds(0, FLUSH)],
                                bdst_hbm.at[pl.ds(ob, FLUSH)])
                vs = osrc[pl.ds(FLUSH, 16)]
                vd = odst[pl.ds(FLUSH, 16)]
                osrc[pl.ds(0, 16)] = vs
                odst[pl.ds(0, 16)] = vd
                return (off - FLUSH, flushed + FLUSH)

            return lax.cond(off >= FLUSH, flush, lambda c3: c3,
                            (off, flushed))

        return lax.fori_loop(0, CHB // 16, vec_body, carry)

    off, flushed = lax.fori_loop(0, E // CHB, chunk_body, (0, 0))
    ob = pl.multiple_of(w * EPAD + flushed, FLUSH)
    pltpu.sync_copy(osrc.at[pl.ds(0, FLUSH)], bsrc_hbm.at[pl.ds(ob, FLUSH)])
    pltpu.sync_copy(odst.at[pl.ds(0, FLUSH)], bdst_hbm.at[pl.ds(ob, FLUSH)])
    cbuf[...] = jnp.full((16,), flushed + off, jnp.int32)
    pltpu.sync_copy(cbuf, cnts_hbm.at[w])


@jax.jit
def _bucket(src, dst):
    f = pl.kernel(
        _bucket_body,
        out_type=[
            jax.ShapeDtypeStruct((_NW * EPAD,), jnp.int32),
            jax.ShapeDtypeStruct((_NW * EPAD,), jnp.int32),
            jax.ShapeDtypeStruct((_NW, 16), jnp.int32),
        ],
        mesh=_sc_mesh,
        compiler_params=pltpu.CompilerParams(needs_layout_passes=False, use_tc_tiling_on_sc=False),
        scratch_types=[
            pltpu.VMEM((CHB,), jnp.int32),
            pltpu.VMEM((CHB,), jnp.int32),
            pltpu.VMEM((FLUSH + 16,), jnp.int32),
            pltpu.VMEM((FLUSH + 16,), jnp.int32),
            pltpu.VMEM((16,), jnp.int32),
        ],
    )
    return f(src, dst)


def _segmax_body(beff_hbm, bsrc_hbm, bdst_hbm, cnts_hbm, m_hbm,
                 idxb, dstb, rows, acc, cbuf, sem):
    w = _worker_id()
    lo = w * RANGE
    iota = lax.iota(jnp.int32, 16)

    pltpu.sync_copy(cnts_hbm.at[w], cbuf)
    cnt = jnp.max(cbuf[...])

    def init_body(r, _):
        for j in range(4):
            acc[r, pl.ds(j * 16, 16)] = jnp.full((16,), _NEG, jnp.float32)
        return 0
    lax.fori_loop(0, ACCR, init_body, 0)

    nch = (cnt + (CHE - 1)) >> 9

    def chunk(ci, _):
        base = pl.multiple_of(w * EPAD + ci * CHE, CHE)
        pltpu.sync_copy(bsrc_hbm.at[pl.ds(base, CHE)], idxb)
        pltpu.sync_copy(bdst_hbm.at[pl.ds(base, CHE)], dstb)
        pltpu.async_copy(beff_hbm.at[idxb], rows, sem).wait()

        def grp(g, _):
            for e in range(16):
                ei = g * 16 + e
                de = plsc.load_gather(dstb, [jnp.full((16,), ei, jnp.int32)])
                for j in range(4):
                    rv = rows[ei, pl.ds(j * 16, 16)]
                    ci_ = iota + (j * 16)
                    cur = plsc.load_gather(acc, [de, ci_])
                    plsc.store_scatter(acc, [de, ci_], jnp.maximum(cur, rv))
            return 0
        lax.fori_loop(0, CHE // 16, grp, 0)
        return 0

    lax.fori_loop(0, nch, chunk, 0)

    pltpu.sync_copy(acc.at[pl.ds(0, RANGE)], m_hbm.at[pl.ds(lo, RANGE)])


@jax.jit
def _segmax(beff, bsrc, bdst, cnts):
    f = pl.kernel(
        _segmax_body,
        out_type=jax.ShapeDtypeStruct((NPAD, H), jnp.float32),
        mesh=_sc_mesh,
        compiler_params=pltpu.CompilerParams(needs_layout_passes=False, use_tc_tiling_on_sc=False),
        scratch_types=[
            pltpu.VMEM((CHE,), jnp.int32),
            pltpu.VMEM((CHE,), jnp.int32),
            pltpu.VMEM((CHE, H), jnp.float32),
            pltpu.VMEM((ACCR, H), jnp.float32),
            pltpu.VMEM((16,), jnp.int32),
            pltpu.SemaphoreType.DMA,
        ],
    )
    return f(beff, bsrc, bdst, cnts)


_BR = 2000


def _tc_call(body, n_in, n_out_cols):
    in_specs = ([pl.BlockSpec((_BR, None), lambda i: (i, 0))] +
                [pl.BlockSpec(lambda i: (0, 0))] * (n_in - 1))
    return body, in_specs


def _first_k(x_ref, w0_ref, c0_ref, wc_ref, xn_ref, a_ref, b_ref):
    xn = jax.nn.relu(jnp.dot(x_ref[...], w0_ref[...],
                             preferred_element_type=jnp.float32) + c0_ref[...])
    p = jnp.dot(xn, wc_ref[...], preferred_element_type=jnp.float32)
    xn_ref[...] = xn
    a_ref[...] = p[:, :H]
    b_ref[...] = p[:, H:]


def _comb_sub_k(xnp_ref, a_ref, m_ref, cv_ref, ga_ref, wc_ref,
                xn_ref, a2_ref, b2_ref):
    agg = jax.nn.relu(a_ref[...] + cv_ref[...] + ga_ref[...] * m_ref[...])
    xn = xnp_ref[...] - agg
    p = jnp.dot(xn, wc_ref[...], preferred_element_type=jnp.float32)
    xn_ref[...] = xn
    a2_ref[...] = p[:, :H]
    b2_ref[...] = p[:, H:]


def _comb_set_k(a_ref, m_ref, cv_ref, ga_ref, wc_ref,
                xn_ref, a2_ref, b2_ref):
    xn = jax.nn.relu(a_ref[...] + cv_ref[...] + ga_ref[...] * m_ref[...])
    p = jnp.dot(xn, wc_ref[...], preferred_element_type=jnp.float32)
    xn_ref[...] = xn
    a2_ref[...] = p[:, :H]
    b2_ref[...] = p[:, H:]


def _final_k(xnp_ref, a_ref, m_ref, cv_ref, ga_ref, w1_ref, c1_ref, o_ref):
    agg = jax.nn.relu(a_ref[...] + cv_ref[...] + ga_ref[...] * m_ref[...])
    xn = xnp_ref[...] - agg
    o = jax.nn.relu(jnp.dot(xn, w1_ref[...],
                            preferred_element_type=jnp.float32) + c1_ref[...])
    mx = jnp.max(o, axis=1, keepdims=True)
    ex = jnp.exp(o - mx)
    s = jnp.sum(ex, axis=1, keepdims=True)
    o_ref[...] = o - mx - jnp.log(s)


def _row_spec(cols):
    return pl.BlockSpec((_BR, cols), lambda i: (i, 0))


def _full_spec(r, c):
    return pl.BlockSpec((r, c), lambda i: (0, 0))


def _tc_first(x, w0, c0, wc):
    return pl.pallas_call(
        _first_k,
        grid=(N // _BR,),
        in_specs=[_row_spec(NIN), _full_spec(NIN, H), _full_spec(1, H),
                  _full_spec(H, 2 * H)],
        out_specs=[_row_spec(H), _row_spec(H), _row_spec(H)],
        out_shape=[jax.ShapeDtypeStruct((N, H), jnp.float32)] * 3,
    )(x, w0, c0, wc)


def _tc_comb_sub(xnp, a, m, cv, ga, wc):
    return pl.pallas_call(
        _comb_sub_k,
        grid=(N // _BR,),
        in_specs=[_row_spec(H), _row_spec(H), _row_spec(H),
                  _full_spec(1, H), _full_spec(1, H), _full_spec(H, 2 * H)],
        out_specs=[_row_spec(H), _row_spec(H), _row_spec(H)],
        out_shape=[jax.ShapeDtypeStruct((N, H), jnp.float32)] * 3,
    )(xnp, a, m, cv, ga, wc)


def _tc_comb_set(a, m, cv, ga, wc):
    return pl.pallas_call(
        _comb_set_k,
        grid=(N // _BR,),
        in_specs=[_row_spec(H), _row_spec(H),
                  _full_spec(1, H), _full_spec(1, H), _full_spec(H, 2 * H)],
        out_specs=[_row_spec(H), _row_spec(H), _row_spec(H)],
        out_shape=[jax.ShapeDtypeStruct((N, H), jnp.float32)] * 3,
    )(a, m, cv, ga, wc)


def _tc_final(xnp, a, m, cv, ga, w1, c1):
    return pl.pallas_call(
        _final_k,
        grid=(N // _BR,),
        in_specs=[_row_spec(H), _row_spec(H), _row_spec(H),
                  _full_spec(1, H), _full_spec(1, H), _full_spec(H, H),
                  _full_spec(1, H)],
        out_specs=_row_spec(H),
        out_shape=jax.ShapeDtypeStruct((N, H), jnp.float32),
    )(xnp, a, m, cv, ga, w1, c1)


def kernel(x, edge_index, lin0_W, lin0_b, lin0_g, lin0_beta, conv_Ws,
           conv_bs, conv_gs, conv_betas, lin1_W, lin1_b, lin1_g, lin1_beta):
    src = edge_index[0]
    dst = edge_index[1]

    sg0 = _BN_S * lin0_g
    w0 = lin0_W * sg0[None, :]
    c0 = (lin0_b * sg0 + lin0_beta)[None, :]
    sg1 = _BN_S * lin1_g
    w1 = lin1_W * sg1[None, :]
    c1 = (lin1_b * sg1 + lin1_beta)[None, :]

    wcs, cvs, gas = [], [], []
    for l in range(4):
        sg = _BN_S * conv_gs[l]
        sgn = jnp.where(sg >= 0, 1.0, -1.0)
        wt, wb = conv_Ws[l][:H], conv_Ws[l][H:]
        wcs.append(jnp.concatenate([(wt - wb) * sg[None, :],
                                    wb * sgn[None, :]], axis=1))
        cvs.append((conv_bs[l] * sg + conv_betas[l])[None, :])
        gas.append(jnp.abs(sg)[None, :])

    bsrc, bdst, cnts = _bucket(src, dst)

    xn0, a0, b0 = _tc_first(x, w0, c0, wcs[0])
    m0 = _segmax(b0, bsrc, bdst, cnts)[:N]
    xn1, a1, b1 = _tc_comb_sub(xn0, a0, m0, cvs[0], gas[0], wcs[1])
    m1 = _segmax(b1, bsrc, bdst, cnts)[:N]
    xn2, a2, b2 = _tc_comb_set(a1, m1, cvs[1], gas[1], wcs[2])
    m2 = _segmax(b2, bsrc, bdst, cnts)[:N]
    xn3, a3, b3 = _tc_comb_sub(xn2, a2, m2, cvs[2], gas[2], wcs[3])
    m3 = _segmax(b3, bsrc, bdst, cnts)[:N]
    return _tc_final(xn3, a3, m3, cvs[3], gas[3], w1, c1)


# speedup vs baseline: 10.4944x; 2.4647x over previous
import functools
import jax
import jax.numpy as jnp
from jax import lax
from jax.experimental import pallas as pl
from jax.experimental.pallas import tpu as pltpu
from jax.experimental.pallas import tpu_sc as plsc

N = 10000
E = 320000
NIN = 128
H = 64
_BN_S = (1.0 + 1e-5) ** -0.5
_NEG = -3.4e38

_NC = 2
_NS = 16
_NW = _NC * _NS
RANGE = 320
NPAD = _NW * RANGE
ACCR = 328
FLUSH = 8192
EPAD = 40 * FLUSH
CHB = 8000
CHE = 512

_sc_mesh = plsc.VectorSubcoreMesh(core_axis_name="c", subcore_axis_name="s",
                                  num_cores=_NC, num_subcores=_NS)


def _worker_id():
    return lax.axis_index("s") * _NC + lax.axis_index("c")


NCHB = E // CHB
SUBI = 125
OCAP = FLUSH + SUBI * 16 + 16


def _bucket_body(src_hbm, dst_hbm, bsrc_hbm, bdst_hbm, cnts_hbm,
                 sb0, sb1, db0, db1, osrc, odst, cbuf, semb0, semb1):
    w = _worker_id()
    lo = w * RANGE
    iota = lax.iota(jnp.int32, 16)
    sbuf = (sb0, sb1)
    dbuf = (db0, db1)
    semb = (semb0, semb1)

    def init_body(i, _):
        base = i * 16
        osrc[pl.ds(base, 16)] = (base + iota) & 8191
        odst[pl.ds(base, 16)] = jnp.full((16,), RANGE, jnp.int32)
        return 0
    lax.fori_loop(0, OCAP // 16, init_body, 0)

    def load_chunk(ci, b):
        cb = pl.multiple_of(ci * CHB, 8)
        pltpu.async_copy(dst_hbm.at[pl.ds(cb, CHB)], dbuf[b], semb[b])
        pltpu.async_copy(src_hbm.at[pl.ds(cb, CHB)], sbuf[b], semb[b])

    def wait_chunk(ci, b):
        cb = pl.multiple_of(ci * CHB, 8)
        pltpu.make_async_copy(dst_hbm.at[pl.ds(cb, CHB)], dbuf[b],
                              semb[b]).wait()
        pltpu.make_async_copy(src_hbm.at[pl.ds(cb, CHB)], sbuf[b],
                              semb[b]).wait()

    load_chunk(0, 0)

    def maybe_flush(offv, flushed):
        def flush(c3):
            offv, flushed = c3
            ob = pl.multiple_of(w * EPAD + flushed, FLUSH)
            pltpu.sync_copy(osrc.at[pl.ds(0, FLUSH)],
                            bsrc_hbm.at[pl.ds(ob, FLUSH)])
            pltpu.sync_copy(odst.at[pl.ds(0, FLUSH)],
                            bdst_hbm.at[pl.ds(ob, FLUSH)])
            def mv(k, _):
                vs = osrc[pl.ds(FLUSH + k * 16, 16)]
                vd = odst[pl.ds(FLUSH + k * 16, 16)]
                osrc[pl.ds(k * 16, 16)] = vs
                odst[pl.ds(k * 16, 16)] = vd
                return 0
            lax.fori_loop(0, SUBI + 1, mv, 0)
            return (offv - FLUSH, flushed + FLUSH)
        return lax.cond(jnp.max(offv) >= FLUSH, flush, lambda c3: c3,
                        (offv, flushed))

    def chunk_body(ci, b, carry):
        offv, flushed = carry

        def sub_body(si, c2):
            offv = c2

            def vec_body(i, offv):
                p = si * SUBI * 16 + i * 16
                d = dbuf[b][pl.ds(p, 16)]
                sv = sbuf[b][pl.ds(p, 16)]
                m = (d >= lo) & (d < lo + RANGE)
                cs = plsc.cumsum(jnp.where(m, jnp.int32(1), jnp.int32(0)))
                idx = (offv + cs) - 1
                plsc.store_scatter(odst, [idx], d - lo, mask=m)
                plsc.store_scatter(osrc, [idx], sv, mask=m)
                return offv + plsc.all_reduce_population_count(m)

            return lax.fori_loop(0, SUBI, vec_body, offv)

        for si in range(CHB // (SUBI * 16)):
            offv = sub_body(si, offv)
            offv, flushed = maybe_flush(offv, flushed)
        return (offv, flushed)

    def outer(co, carry):
        for b in range(2):
            ci = co * 2 + b

            @pl.when(ci + 1 < NCHB)
            def _():
                load_chunk(ci + 1, b ^ 1)

            wait_chunk(ci, b)
            carry = chunk_body(ci, b, carry)
        return carry

    offv, flushed = lax.fori_loop(0, NCHB // 2, outer,
                                  (jnp.zeros((16,), jnp.int32), 0))
    ob = pl.multiple_of(w * EPAD + flushed, FLUSH)
    pltpu.sync_copy(osrc.at[pl.ds(0, FLUSH)], bsrc_hbm.at[pl.ds(ob, FLUSH)])
    pltpu.sync_copy(odst.at[pl.ds(0, FLUSH)], bdst_hbm.at[pl.ds(ob, FLUSH)])
    cbuf[...] = jnp.full((16,), flushed + jnp.max(offv), jnp.int32)
    pltpu.sync_copy(cbuf, cnts_hbm.at[w])


@jax.jit
def _bucket(src, dst):
    f = pl.kernel(
        _bucket_body,
        out_type=[
            jax.ShapeDtypeStruct((_NW * EPAD,), jnp.int32),
            jax.ShapeDtypeStruct((_NW * EPAD,), jnp.int32),
            jax.ShapeDtypeStruct((_NW, 16), jnp.int32),
        ],
        mesh=_sc_mesh,
        compiler_params=pltpu.CompilerParams(needs_layout_passes=False, use_tc_tiling_on_sc=False),
        scratch_types=[
            pltpu.VMEM((CHB,), jnp.int32),
            pltpu.VMEM((CHB,), jnp.int32),
            pltpu.VMEM((CHB,), jnp.int32),
            pltpu.VMEM((CHB,), jnp.int32),
            pltpu.VMEM((OCAP,), jnp.int32),
            pltpu.VMEM((OCAP,), jnp.int32),
            pltpu.VMEM((16,), jnp.int32),
            pltpu.SemaphoreType.DMA,
            pltpu.SemaphoreType.DMA,
        ],
    )
    return f(src, dst)


_DN = lax.GatherDimensionNumbers(offset_dims=(), collapsed_slice_dims=(0,),
                                 start_index_map=(0,))


def _lane(v, e):
    return lax.gather(v, jnp.full((16, 1), e, jnp.int32), _DN, (1,),
                      mode=lax.GatherScatterMode.PROMISE_IN_BOUNDS)


def _segmax_body(beff_hbm, bsrc_hbm, bdst_hbm, cnts_hbm, m_hbm,
                 idx0, idx1, dst0, dst1, rows0, rows1,
                 acc0, acc1, acc2, acc3, mstage, cbuf,
                 semr0, semr1):
    w = _worker_id()
    lo = w * RANGE
    sid = lax.axis_index("s")
    iota = lax.iota(jnp.int32, 16)
    idxb = (idx0, idx1)
    dstb = (dst0, dst1)
    rows = (rows0, rows1)
    semr = (semr0, semr1)
    accs = (acc0, acc1, acc2, acc3)

    pltpu.sync_copy(cnts_hbm.at[w], cbuf)
    cnt = jnp.max(cbuf[...])

    neg = jnp.full((16,), _NEG, jnp.float32)

    def init_body(r, _):
        for aj in accs:
            aj[r, :] = neg
        return 0
    lax.fori_loop(0, ACCR, init_body, 0)

    nch = (cnt + (CHE - 1)) >> 9

    def load_chunk(ci, b):
        base = pl.multiple_of(w * EPAD + ci * CHE, CHE)
        pltpu.sync_copy(bsrc_hbm.at[pl.ds(base, CHE)], idxb[b])
        pltpu.sync_copy(bdst_hbm.at[pl.ds(base, CHE)], dstb[b])

    @pl.when(nch > 0)
    def _():
        load_chunk(0, 0)
        pltpu.async_copy(beff_hbm.at[idx0], rows0, semr0)

    def compute(b):
        rb = rows[b]
        db = dstb[b]

        def grp(g, _):
            dv = db[pl.ds(g * 16, 16)]
            for e in range(16):
                ei = g * 16 + e
                de = _lane(dv, e)
                rvs = [rb[ei, pl.ds(j * 16, 16)] for j in range(4)]
                curs = [plsc.load_gather(aj, [de, iota]) for aj in accs]
                res = [jnp.maximum(c, r) for c, r in zip(curs, rvs)]
                for j, aj in enumerate(accs):
                    plsc.store_scatter(aj, [de, iota], res[j])
            return 0
        lax.fori_loop(0, CHE // 16, grp, 0)

    def outer(co, _):
        for b in range(2):
            ci = co * 2 + b

            @pl.when(ci + 1 < nch)
            def _():
                load_chunk(ci + 1, b ^ 1)
                pltpu.async_copy(beff_hbm.at[idxb[b ^ 1]], rows[b ^ 1],
                                 semr[b ^ 1])

            @pl.when(ci < nch)
            def _():
                pltpu.make_async_copy(beff_hbm.at[idxb[b]], rows[b],
                                      semr[b]).wait()
                compute(b)
        return 0

    lax.fori_loop(0, (nch + 1) >> 1, outer, 0)

    def out_body(r, _):
        for j, aj in enumerate(accs):
            mstage[r, pl.ds(j * 16, 16)] = aj[r, :]
        return 0
    lax.fori_loop(0, RANGE, out_body, 0)
    pltpu.sync_copy(mstage, m_hbm.at[pl.ds(lo, RANGE)])


@jax.jit
def _segmax(beff, bsrc, bdst, cnts):
    f = pl.kernel(
        _segmax_body,
        out_type=jax.ShapeDtypeStruct((NPAD, H), jnp.float32),
        mesh=_sc_mesh,
        compiler_params=pltpu.CompilerParams(needs_layout_passes=False, use_tc_tiling_on_sc=False),
        scratch_types=[
            pltpu.VMEM((CHE,), jnp.int32),
            pltpu.VMEM((CHE,), jnp.int32),
            pltpu.VMEM((CHE,), jnp.int32),
            pltpu.VMEM((CHE,), jnp.int32),
            pltpu.VMEM((CHE, H), jnp.float32),
            pltpu.VMEM((CHE, H), jnp.float32),
            pltpu.VMEM((ACCR, 16), jnp.float32),
            pltpu.VMEM((ACCR, 16), jnp.float32),
            pltpu.VMEM((ACCR, 16), jnp.float32),
            pltpu.VMEM((ACCR, 16), jnp.float32),
            pltpu.VMEM((RANGE, H), jnp.float32),
            pltpu.VMEM((16,), jnp.int32),
            pltpu.SemaphoreType.DMA,
            pltpu.SemaphoreType.DMA,
        ],
    )
    return f(beff, bsrc, bdst, cnts)


_BR = 2000


def _tc_call(body, n_in, n_out_cols):
    in_specs = ([pl.BlockSpec((_BR, None), lambda i: (i, 0))] +
                [pl.BlockSpec(lambda i: (0, 0))] * (n_in - 1))
    return body, in_specs


def _first_k(x_ref, w0_ref, c0_ref, wc_ref, xn_ref, a_ref, b_ref):
    xn = jax.nn.relu(jnp.dot(x_ref[...], w0_ref[...],
                             preferred_element_type=jnp.float32) + c0_ref[...])
    p = jnp.dot(xn, wc_ref[...], preferred_element_type=jnp.float32)
    xn_ref[...] = xn
    a_ref[...] = p[:, :H]
    b_ref[...] = p[:, H:]


def _comb_sub_k(xnp_ref, a_ref, m_ref, cv_ref, ga_ref, wc_ref,
                xn_ref, a2_ref, b2_ref):
    agg = jax.nn.relu(a_ref[...] + cv_ref[...] + ga_ref[...] * m_ref[...])
    xn = xnp_ref[...] - agg
    p = jnp.dot(xn, wc_ref[...], preferred_element_type=jnp.float32)
    xn_ref[...] = xn
    a2_ref[...] = p[:, :H]
    b2_ref[...] = p[:, H:]


def _comb_set_k(a_ref, m_ref, cv_ref, ga_ref, wc_ref,
                xn_ref, a2_ref, b2_ref):
    xn = jax.nn.relu(a_ref[...] + cv_ref[...] + ga_ref[...] * m_ref[...])
    p = jnp.dot(xn, wc_ref[...], preferred_element_type=jnp.float32)
    xn_ref[...] = xn
    a2_ref[...] = p[:, :H]
    b2_ref[...] = p[:, H:]


def _final_k(xnp_ref, a_ref, m_ref, cv_ref, ga_ref, w1_ref, c1_ref, o_ref):
    agg = jax.nn.relu(a_ref[...] + cv_ref[...] + ga_ref[...] * m_ref[...])
    xn = xnp_ref[...] - agg
    o = jax.nn.relu(jnp.dot(xn, w1_ref[...],
                            preferred_element_type=jnp.float32) + c1_ref[...])
    mx = jnp.max(o, axis=1, keepdims=True)
    ex = jnp.exp(o - mx)
    s = jnp.sum(ex, axis=1, keepdims=True)
    o_ref[...] = o - mx - jnp.log(s)


def _row_spec(cols):
    return pl.BlockSpec((_BR, cols), lambda i: (i, 0))


def _full_spec(r, c):
    return pl.BlockSpec((r, c), lambda i: (0, 0))


def _tc_first(x, w0, c0, wc):
    return pl.pallas_call(
        _first_k,
        grid=(N // _BR,),
        in_specs=[_row_spec(NIN), _full_spec(NIN, H), _full_spec(1, H),
                  _full_spec(H, 2 * H)],
        out_specs=[_row_spec(H), _row_spec(H), _row_spec(H)],
        out_shape=[jax.ShapeDtypeStruct((N, H), jnp.float32)] * 3,
    )(x, w0, c0, wc)


def _tc_comb_sub(xnp, a, m, cv, ga, wc):
    return pl.pallas_call(
        _comb_sub_k,
        grid=(N // _BR,),
        in_specs=[_row_spec(H), _row_spec(H), _row_spec(H),
                  _full_spec(1, H), _full_spec(1, H), _full_spec(H, 2 * H)],
        out_specs=[_row_spec(H), _row_spec(H), _row_spec(H)],
        out_shape=[jax.ShapeDtypeStruct((N, H), jnp.float32)] * 3,
    )(xnp, a, m, cv, ga, wc)


def _tc_comb_set(a, m, cv, ga, wc):
    return pl.pallas_call(
        _comb_set_k,
        grid=(N // _BR,),
        in_specs=[_row_spec(H), _row_spec(H),
                  _full_spec(1, H), _full_spec(1, H), _full_spec(H, 2 * H)],
        out_specs=[_row_spec(H), _row_spec(H), _row_spec(H)],
        out_shape=[jax.ShapeDtypeStruct((N, H), jnp.float32)] * 3,
    )(a, m, cv, ga, wc)


def _tc_final(xnp, a, m, cv, ga, w1, c1):
    return pl.pallas_call(
        _final_k,
        grid=(N // _BR,),
        in_specs=[_row_spec(H), _row_spec(H), _row_spec(H),
                  _full_spec(1, H), _full_spec(1, H), _full_spec(H, H),
                  _full_spec(1, H)],
        out_specs=_row_spec(H),
        out_shape=jax.ShapeDtypeStruct((N, H), jnp.float32),
    )(xnp, a, m, cv, ga, w1, c1)


def kernel(x, edge_index, lin0_W, lin0_b, lin0_g, lin0_beta, conv_Ws,
           conv_bs, conv_gs, conv_betas, lin1_W, lin1_b, lin1_g, lin1_beta):
    src = edge_index[0]
    dst = edge_index[1]

    sg0 = _BN_S * lin0_g
    w0 = lin0_W * sg0[None, :]
    c0 = (lin0_b * sg0 + lin0_beta)[None, :]
    sg1 = _BN_S * lin1_g
    w1 = lin1_W * sg1[None, :]
    c1 = (lin1_b * sg1 + lin1_beta)[None, :]

    wcs, cvs, gas = [], [], []
    for l in range(4):
        sg = _BN_S * conv_gs[l]
        sgn = jnp.where(sg >= 0, 1.0, -1.0)
        wt, wb = conv_Ws[l][:H], conv_Ws[l][H:]
        wcs.append(jnp.concatenate([(wt - wb) * sg[None, :],
                                    wb * sgn[None, :]], axis=1))
        cvs.append((conv_bs[l] * sg + conv_betas[l])[None, :])
        gas.append(jnp.abs(sg)[None, :])

    bsrc, bdst, cnts = _bucket(src, dst)

    def pad(b):
        return jnp.pad(b, ((0, NPAD - N), (0, 0)))

    xn0, a0, b0 = _tc_first(x, w0, c0, wcs[0])
    m0 = _segmax(pad(b0), bsrc, bdst, cnts)[:N]
    xn1, a1, b1 = _tc_comb_sub(xn0, a0, m0, cvs[0], gas[0], wcs[1])
    m1 = _segmax(pad(b1), bsrc, bdst, cnts)[:N]
    xn2, a2, b2 = _tc_comb_set(a1, m1, cvs[1], gas[1], wcs[2])
    m2 = _segmax(pad(b2), bsrc, bdst, cnts)[:N]
    xn3, a3, b3 = _tc_comb_sub(xn2, a2, m2, cvs[2], gas[2], wcs[3])
    m3 = _segmax(pad(b3), bsrc, bdst, cnts)[:N]
    return _tc_final(xn3, a3, m3, cvs[3], gas[3], w1, c1)


# speedup vs baseline: 11.1893x; 1.0662x over previous
import functools
import jax
import jax.numpy as jnp
from jax import lax
from jax.experimental import pallas as pl
from jax.experimental.pallas import tpu as pltpu
from jax.experimental.pallas import tpu_sc as plsc

N = 10000
E = 320000
NIN = 128
H = 64
_BN_S = (1.0 + 1e-5) ** -0.5
_NEG = -3.4e38

_NC = 2
_NS = 16
_NW = _NC * _NS
RANGE = 320
NPAD = _NW * RANGE
ACCR = 328
FLUSH = 8192
EPAD = 40 * FLUSH
CHB = 8000
CAP = 16384
CH2 = 2000
CHE = 512

_sc_mesh = plsc.VectorSubcoreMesh(core_axis_name="c", subcore_axis_name="s",
                                  num_cores=_NC, num_subcores=_NS)


def _worker_id():
    return lax.axis_index("s") * _NC + lax.axis_index("c")


NCHB = E // CHB
SUBI = 125
OCAP = FLUSH + SUBI * 16 + 16


def _bucket_body(src_hbm, dst_hbm, bsrc_hbm, bdst_hbm, cnts_hbm,
                 bsrc2_hbm, bdst2_hbm, cnts2_hbm,
                 sb0, sb1, db0, db1, osrc, odst, cbuf,
                 hist, offs, srts, srtd, semb0, semb1):
    w = _worker_id()
    lo = w * RANGE
    iota = lax.iota(jnp.int32, 16)
    sbuf = (sb0, sb1)
    dbuf = (db0, db1)
    semb = (semb0, semb1)

    def init_body(i, _):
        base = i * 16
        osrc[pl.ds(base, 16)] = (base + iota) & 8191
        odst[pl.ds(base, 16)] = jnp.full((16,), RANGE, jnp.int32)
        return 0
    lax.fori_loop(0, OCAP // 16, init_body, 0)

    def load_chunk(ci, b):
        cb = pl.multiple_of(ci * CHB, 8)
        pltpu.async_copy(dst_hbm.at[pl.ds(cb, CHB)], dbuf[b], semb[b])
        pltpu.async_copy(src_hbm.at[pl.ds(cb, CHB)], sbuf[b], semb[b])

    def wait_chunk(ci, b):
        cb = pl.multiple_of(ci * CHB, 8)
        pltpu.make_async_copy(dst_hbm.at[pl.ds(cb, CHB)], dbuf[b],
                              semb[b]).wait()
        pltpu.make_async_copy(src_hbm.at[pl.ds(cb, CHB)], sbuf[b],
                              semb[b]).wait()

    load_chunk(0, 0)

    def maybe_flush(offv, flushed):
        def flush(c3):
            offv, flushed = c3
            ob = pl.multiple_of(w * EPAD + flushed, FLUSH)
            pltpu.sync_copy(osrc.at[pl.ds(0, FLUSH)],
                            bsrc_hbm.at[pl.ds(ob, FLUSH)])
            pltpu.sync_copy(odst.at[pl.ds(0, FLUSH)],
                            bdst_hbm.at[pl.ds(ob, FLUSH)])
            def mv(k, _):
                vs = osrc[pl.ds(FLUSH + k * 16, 16)]
                vd = odst[pl.ds(FLUSH + k * 16, 16)]
                osrc[pl.ds(k * 16, 16)] = vs
                odst[pl.ds(k * 16, 16)] = vd
                return 0
            lax.fori_loop(0, SUBI + 1, mv, 0)
            return (offv - FLUSH, flushed + FLUSH)
        return lax.cond(jnp.max(offv) >= FLUSH, flush, lambda c3: c3,
                        (offv, flushed))

    def chunk_body(ci, b, carry):
        offv, flushed = carry

        def sub_body(si, c2):
            offv = c2

            def vec_body(i, offv):
                for u in range(5):
                    p = si * SUBI * 16 + (i * 5 + u) * 16
                    d = dbuf[b][pl.ds(p, 16)]
                    sv = sbuf[b][pl.ds(p, 16)]
                    m = (d >= lo) & (d < lo + RANGE)
                    cs = plsc.cumsum(jnp.where(m, jnp.int32(1), jnp.int32(0)))
                    idx = (offv + cs) - 1
                    plsc.store_scatter(odst, [idx], d - lo, mask=m)
                    plsc.store_scatter(osrc, [idx], sv, mask=m)
                    offv = offv + plsc.all_reduce_population_count(m)
                return offv

            return lax.fori_loop(0, SUBI // 5, vec_body, offv)

        for si in range(CHB // (SUBI * 16)):
            offv = sub_body(si, offv)
            offv, flushed = maybe_flush(offv, flushed)
        return (offv, flushed)

    def outer(co, carry):
        for b in range(2):
            ci = co * 2 + b

            @pl.when(ci + 1 < NCHB)
            def _():
                load_chunk(ci + 1, b ^ 1)

            wait_chunk(ci, b)
            carry = chunk_body(ci, b, carry)
        return carry

    offv, flushed = lax.fori_loop(0, NCHB // 2, outer,
                                  (jnp.zeros((16,), jnp.int32), 0))
    ob = pl.multiple_of(w * EPAD + flushed, FLUSH)
    pltpu.sync_copy(osrc.at[pl.ds(0, FLUSH)], bsrc_hbm.at[pl.ds(ob, FLUSH)])
    pltpu.sync_copy(odst.at[pl.ds(0, FLUSH)], bdst_hbm.at[pl.ds(ob, FLUSH)])
    cnt = flushed + jnp.max(offv)
    cbuf[...] = jnp.full((16,), cnt, jnp.int32)
    pltpu.sync_copy(cbuf, cnts_hbm.at[w])

    nc2 = (cnt + (CH2 - 1)) // CH2
    cnt2 = nc2 * CH2
    ok = cnt2 <= CAP

    @pl.when(ok)
    def _():
        def hz(r, _):
            hist[r, :] = jnp.zeros((16,), jnp.int32)
            return 0
        lax.fori_loop(0, ACCR, hz, 0)

        ones = jnp.full((16,), 1, jnp.int32)

        def hchunk(k, _):
            base = pl.multiple_of(w * EPAD + k * CH2, 8)
            pltpu.sync_copy(bdst_hbm.at[pl.ds(base, CH2)],
                            db0.at[pl.ds(0, CH2)])

            def hv(i, _):
                dv = db0[pl.ds(i * 16, 16)]
                plsc.addupdate_scatter(hist, [dv, iota], ones)
                return 0
            lax.fori_loop(0, CH2 // 16, hv, 0)
            return 0
        lax.fori_loop(0, nc2, hchunk, 0)

        def pz(r, carry):
            hrow = hist[r, :]
            inc = plsc.cumsum(hrow)
            offs[r, :] = (carry + inc) - hrow
            return carry + _lane(inc, 15)
        lax.fori_loop(0, ACCR, pz, jnp.zeros((16,), jnp.int32))

        def sf(i, _):
            base = i * 16
            srts[pl.ds(base, 16)] = (base + iota) & 8191
            srtd[pl.ds(base, 16)] = jnp.full((16,), RANGE, jnp.int32)
            return 0
        lax.fori_loop(0, CAP // 16, sf, 0)

        def schunk(k, _):
            base = pl.multiple_of(w * EPAD + k * CH2, 8)
            pltpu.sync_copy(bdst_hbm.at[pl.ds(base, CH2)],
                            db0.at[pl.ds(0, CH2)])
            pltpu.sync_copy(bsrc_hbm.at[pl.ds(base, CH2)],
                            sb0.at[pl.ds(0, CH2)])

            def sv2(i, _):
                dv = db0[pl.ds(i * 16, 16)]
                sv = sb0[pl.ds(i * 16, 16)]
                pos = plsc.load_gather(offs, [dv, iota])
                plsc.store_scatter(offs, [dv, iota], pos + 1)
                plsc.store_scatter(srts, [pos], sv)
                plsc.store_scatter(srtd, [pos], dv)
                return 0
            lax.fori_loop(0, CH2 // 16, sv2, 0)
            return 0
        lax.fori_loop(0, nc2, schunk, 0)

        ob2 = pl.multiple_of(w * CAP, 8)
        pltpu.sync_copy(srts, bsrc2_hbm.at[pl.ds(ob2, CAP)])
        pltpu.sync_copy(srtd, bdst2_hbm.at[pl.ds(ob2, CAP)])

    cbuf[...] = jnp.full((16,), jnp.where(ok, cnt2, -1), jnp.int32)
    pltpu.sync_copy(cbuf, cnts2_hbm.at[w])


@jax.jit
def _bucket(src, dst):
    f = pl.kernel(
        _bucket_body,
        out_type=[
            jax.ShapeDtypeStruct((_NW * EPAD,), jnp.int32),
            jax.ShapeDtypeStruct((_NW * EPAD,), jnp.int32),
            jax.ShapeDtypeStruct((_NW, 16), jnp.int32),
            jax.ShapeDtypeStruct((_NW * CAP,), jnp.int32),
            jax.ShapeDtypeStruct((_NW * CAP,), jnp.int32),
            jax.ShapeDtypeStruct((_NW, 16), jnp.int32),
        ],
        mesh=_sc_mesh,
        compiler_params=pltpu.CompilerParams(needs_layout_passes=False, use_tc_tiling_on_sc=False),
        scratch_types=[
            pltpu.VMEM((CHB,), jnp.int32),
            pltpu.VMEM((CHB,), jnp.int32),
            pltpu.VMEM((CHB,), jnp.int32),
            pltpu.VMEM((CHB,), jnp.int32),
            pltpu.VMEM((OCAP,), jnp.int32),
            pltpu.VMEM((OCAP,), jnp.int32),
            pltpu.VMEM((16,), jnp.int32),
            pltpu.VMEM((ACCR, 16), jnp.int32),
            pltpu.VMEM((ACCR, 16), jnp.int32),
            pltpu.VMEM((CAP,), jnp.int32),
            pltpu.VMEM((CAP,), jnp.int32),
            pltpu.SemaphoreType.DMA,
            pltpu.SemaphoreType.DMA,
        ],
    )
    return f(src, dst)


_DN = lax.GatherDimensionNumbers(offset_dims=(), collapsed_slice_dims=(0,),
                                 start_index_map=(0,))


def _lane(v, e):
    return lax.gather(v, jnp.full((16, 1), e, jnp.int32), _DN, (1,),
                      mode=lax.GatherScatterMode.PROMISE_IN_BOUNDS)


def _segmax_body(beff_hbm, bsrc_hbm, bdst_hbm, cnts_hbm,
                 bsrc2_hbm, bdst2_hbm, cnts2_hbm, m_hbm,
                 idx0, idx1, dst0, dst1, rows0, rows1,
                 acc0, acc1, acc2, acc3, mstage, cbuf,
                 semr0, semr1):
    w = _worker_id()
    lo = w * RANGE
    sid = lax.axis_index("s")
    iota = lax.iota(jnp.int32, 16)
    idxb = (idx0, idx1)
    dstb = (dst0, dst1)
    rows = (rows0, rows1)
    semr = (semr0, semr1)
    accs = (acc0, acc1, acc2, acc3)

    pltpu.sync_copy(cnts2_hbm.at[w], cbuf)
    cnt2 = jnp.max(cbuf[...])
    sorted_ok = cnt2 >= 0
    pltpu.sync_copy(cnts_hbm.at[w], cbuf)
    cnt_raw = jnp.max(cbuf[...])
    cnt = jnp.where(sorted_ok, cnt2, cnt_raw)

    neg = jnp.full((16,), _NEG, jnp.float32)

    def init_body(r, _):
        for aj in accs:
            aj[r, :] = neg
        return 0
    lax.fori_loop(0, ACCR, init_body, 0)

    nch = (cnt + (CHE - 1)) >> 9

    def load_chunk(ci, b):
        @pl.when(sorted_ok)
        def _():
            base = pl.multiple_of(w * CAP + ci * CHE, CHE)
            pltpu.sync_copy(bsrc2_hbm.at[pl.ds(base, CHE)], idxb[b])
            pltpu.sync_copy(bdst2_hbm.at[pl.ds(base, CHE)], dstb[b])

        @pl.when(jnp.logical_not(sorted_ok))
        def _():
            base = pl.multiple_of(w * EPAD + ci * CHE, CHE)
            pltpu.sync_copy(bsrc_hbm.at[pl.ds(base, CHE)], idxb[b])
            pltpu.sync_copy(bdst_hbm.at[pl.ds(base, CHE)], dstb[b])

    @pl.when(nch > 0)
    def _():
        load_chunk(0, 0)
        pltpu.async_copy(beff_hbm.at[idx0], rows0, semr0)

    def compute_rmw(b):
        rb = rows[b]
        db = dstb[b]

        def grp(g, _):
            dv = db[pl.ds(g * 16, 16)]
            for e in range(16):
                ei = g * 16 + e
                de = _lane(dv, e)
                rvs = [rb[ei, pl.ds(j * 16, 16)] for j in range(4)]
                curs = [plsc.load_gather(aj, [de, iota]) for aj in accs]
                res = [jnp.maximum(c, r) for c, r in zip(curs, rvs)]
                for j, aj in enumerate(accs):
                    plsc.store_scatter(aj, [de, iota], res[j])
            return 0
        lax.fori_loop(0, CHE // 16, grp, 0)

    def compute_sorted(b, carry):
        rb = rows[b]
        db = dstb[b]

        def grp(g, carry):
            r0, r1, r2, r3, dprev = carry
            rs = [r0, r1, r2, r3]
            dv = db[pl.ds(g * 16, 16)]
            for e in range(16):
                ei = g * 16 + e
                de = _lane(dv, e)
                mask = de != dprev
                rvs = [rb[ei, pl.ds(j * 16, 16)] for j in range(4)]
                for j, aj in enumerate(accs):
                    plsc.store_scatter(aj, [dprev, iota], rs[j], mask=mask)
                rs = [jnp.where(mask, rv, jnp.maximum(r, rv))
                      for r, rv in zip(rs, rvs)]
                dprev = de
            return (rs[0], rs[1], rs[2], rs[3], dprev)
        return lax.fori_loop(0, CHE // 16, grp, carry)

    def outer(co, _):
        for b in range(2):
            ci = co * 2 + b

            @pl.when(ci + 1 < nch)
            def _():
                load_chunk(ci + 1, b ^ 1)
                pltpu.async_copy(beff_hbm.at[idxb[b ^ 1]], rows[b ^ 1],
                                 semr[b ^ 1])

            @pl.when(ci < nch)
            def _():
                pltpu.make_async_copy(beff_hbm.at[idxb[b]], rows[b],
                                      semr[b]).wait()

                @pl.when(jnp.logical_not(sorted_ok))
                def _():
                    compute_rmw(b)
        return 0

    def outer_sorted(co, carry):
        for b in range(2):
            ci = co * 2 + b

            @pl.when(ci + 1 < nch)
            def _():
                load_chunk(ci + 1, b ^ 1)
                pltpu.async_copy(beff_hbm.at[idxb[b ^ 1]], rows[b ^ 1],
                                 semr[b ^ 1])

            def do(c):
                pltpu.make_async_copy(beff_hbm.at[idxb[b]], rows[b],
                                      semr[b]).wait()
                return compute_sorted(b, c)
            carry = lax.cond(ci < nch, do, lambda c: c, carry)
        return carry

    neg4 = (neg, neg, neg, neg, jnp.full((16,), RANGE, jnp.int32))

    @pl.when(sorted_ok)
    def _():
        r0, r1, r2, r3, dprev = lax.fori_loop(0, (nch + 1) >> 1,
                                              outer_sorted, neg4)
        for j, (aj, rj) in enumerate(zip(accs, (r0, r1, r2, r3))):
            plsc.store_scatter(aj, [dprev, iota], rj)

    @pl.when(jnp.logical_not(sorted_ok))
    def _():
        lax.fori_loop(0, (nch + 1) >> 1, outer, 0)

    def out_body(r, _):
        for j, aj in enumerate(accs):
            mstage[r, pl.ds(j * 16, 16)] = aj[r, :]
        return 0
    lax.fori_loop(0, RANGE, out_body, 0)
    pltpu.sync_copy(mstage, m_hbm.at[pl.ds(lo, RANGE)])


@jax.jit
def _segmax(beff, bsrc, bdst, cnts, bsrc2, bdst2, cnts2):
    f = pl.kernel(
        _segmax_body,
        out_type=jax.ShapeDtypeStruct((NPAD, H), jnp.float32),
        mesh=_sc_mesh,
        compiler_params=pltpu.CompilerParams(needs_layout_passes=False, use_tc_tiling_on_sc=False),
        scratch_types=[
            pltpu.VMEM((CHE,), jnp.int32),
            pltpu.VMEM((CHE,), jnp.int32),
            pltpu.VMEM((CHE,), jnp.int32),
            pltpu.VMEM((CHE,), jnp.int32),
            pltpu.VMEM((CHE, H), jnp.float32),
            pltpu.VMEM((CHE, H), jnp.float32),
            pltpu.VMEM((ACCR, 16), jnp.float32),
            pltpu.VMEM((ACCR, 16), jnp.float32),
            pltpu.VMEM((ACCR, 16), jnp.float32),
            pltpu.VMEM((ACCR, 16), jnp.float32),
            pltpu.VMEM((RANGE, H), jnp.float32),
            pltpu.VMEM((16,), jnp.int32),
            pltpu.SemaphoreType.DMA,
            pltpu.SemaphoreType.DMA,
        ],
    )
    return f(beff, bsrc, bdst, cnts, bsrc2, bdst2, cnts2)


_BR = 2000


def _tc_call(body, n_in, n_out_cols):
    in_specs = ([pl.BlockSpec((_BR, None), lambda i: (i, 0))] +
                [pl.BlockSpec(lambda i: (0, 0))] * (n_in - 1))
    return body, in_specs


def _first_k(x_ref, w0_ref, c0_ref, wc_ref, xn_ref, a_ref, b_ref):
    xn = jax.nn.relu(jnp.dot(x_ref[...], w0_ref[...],
                             preferred_element_type=jnp.float32) + c0_ref[...])
    p = jnp.dot(xn, wc_ref[...], preferred_element_type=jnp.float32)
    xn_ref[...] = xn
    a_ref[...] = p[:, :H]
    b_ref[...] = p[:, H:]


def _comb_sub_k(xnp_ref, a_ref, m_ref, cv_ref, ga_ref, wc_ref,
                xn_ref, a2_ref, b2_ref):
    agg = jax.nn.relu(a_ref[...] + cv_ref[...] + ga_ref[...] * m_ref[...])
    xn = xnp_ref[...] - agg
    p = jnp.dot(xn, wc_ref[...], preferred_element_type=jnp.float32)
    xn_ref[...] = xn
    a2_ref[...] = p[:, :H]
    b2_ref[...] = p[:, H:]


def _comb_set_k(a_ref, m_ref, cv_ref, ga_ref, wc_ref,
                xn_ref, a2_ref, b2_ref):
    xn = jax.nn.relu(a_ref[...] + cv_ref[...] + ga_ref[...] * m_ref[...])
    p = jnp.dot(xn, wc_ref[...], preferred_element_type=jnp.float32)
    xn_ref[...] = xn
    a2_ref[...] = p[:, :H]
    b2_ref[...] = p[:, H:]


def _final_k(xnp_ref, a_ref, m_ref, cv_ref, ga_ref, w1_ref, c1_ref, o_ref):
    agg = jax.nn.relu(a_ref[...] + cv_ref[...] + ga_ref[...] * m_ref[...])
    xn = xnp_ref[...] - agg
    o = jax.nn.relu(jnp.dot(xn, w1_ref[...],
                            preferred_element_type=jnp.float32) + c1_ref[...])
    mx = jnp.max(o, axis=1, keepdims=True)
    ex = jnp.exp(o - mx)
    s = jnp.sum(ex, axis=1, keepdims=True)
    o_ref[...] = o - mx - jnp.log(s)


def _row_spec(cols):
    return pl.BlockSpec((_BR, cols), lambda i: (i, 0))


def _full_spec(r, c):
    return pl.BlockSpec((r, c), lambda i: (0, 0))


def _tc_first(x, w0, c0, wc):
    return pl.pallas_call(
        _first_k,
        grid=(N // _BR,),
        in_specs=[_row_spec(NIN), _full_spec(NIN, H), _full_spec(1, H),
                  _full_spec(H, 2 * H)],
        out_specs=[_row_spec(H), _row_spec(H), _row_spec(H)],
        out_shape=[jax.ShapeDtypeStruct((N, H), jnp.float32)] * 3,
    )(x, w0, c0, wc)


def _tc_comb_sub(xnp, a, m, cv, ga, wc):
    return pl.pallas_call(
        _comb_sub_k,
        grid=(N // _BR,),
        in_specs=[_row_spec(H), _row_spec(H), _row_spec(H),
                  _full_spec(1, H), _full_spec(1, H), _full_spec(H, 2 * H)],
        out_specs=[_row_spec(H), _row_spec(H), _row_spec(H)],
        out_shape=[jax.ShapeDtypeStruct((N, H), jnp.float32)] * 3,
    )(xnp, a, m, cv, ga, wc)


def _tc_comb_set(a, m, cv, ga, wc):
    return pl.pallas_call(
        _comb_set_k,
        grid=(N // _BR,),
        in_specs=[_row_spec(H), _row_spec(H),
                  _full_spec(1, H), _full_spec(1, H), _full_spec(H, 2 * H)],
        out_specs=[_row_spec(H), _row_spec(H), _row_spec(H)],
        out_shape=[jax.ShapeDtypeStruct((N, H), jnp.float32)] * 3,
    )(a, m, cv, ga, wc)


def _tc_final(xnp, a, m, cv, ga, w1, c1):
    return pl.pallas_call(
        _final_k,
        grid=(N // _BR,),
        in_specs=[_row_spec(H), _row_spec(H), _row_spec(H),
                  _full_spec(1, H), _full_spec(1, H), _full_spec(H, H),
                  _full_spec(1, H)],
        out_specs=_row_spec(H),
        out_shape=jax.ShapeDtypeStruct((N, H), jnp.float32),
    )(xnp, a, m, cv, ga, w1, c1)


def kernel(x, edge_index, lin0_W, lin0_b, lin0_g, lin0_beta, conv_Ws,
           conv_bs, conv_gs, conv_betas, lin1_W, lin1_b, lin1_g, lin1_beta):
    src = edge_index[0]
    dst = edge_index[1]

    sg0 = _BN_S * lin0_g
    w0 = lin0_W * sg0[None, :]
    c0 = (lin0_b * sg0 + lin0_beta)[None, :]
    sg1 = _BN_S * lin1_g
    w1 = lin1_W * sg1[None, :]
    c1 = (lin1_b * sg1 + lin1_beta)[None, :]

    wcs, cvs, gas = [], [], []
    for l in range(4):
        sg = _BN_S * conv_gs[l]
        sgn = jnp.where(sg >= 0, 1.0, -1.0)
        wt, wb = conv_Ws[l][:H], conv_Ws[l][H:]
        wcs.append(jnp.concatenate([(wt - wb) * sg[None, :],
                                    wb * sgn[None, :]], axis=1))
        cvs.append((conv_bs[l] * sg + conv_betas[l])[None, :])
        gas.append(jnp.abs(sg)[None, :])

    bsrc, bdst, cnts, bsrc2, bdst2, cnts2 = _bucket(src, dst)

    def pad(b):
        return jnp.pad(b, ((0, NPAD - N), (0, 0)))

    xn0, a0, b0 = _tc_first(x, w0, c0, wcs[0])
    m0 = _segmax(pad(b0), bsrc, bdst, cnts, bsrc2, bdst2, cnts2)[:N]
    xn1, a1, b1 = _tc_comb_sub(xn0, a0, m0, cvs[0], gas[0], wcs[1])
    m1 = _segmax(pad(b1), bsrc, bdst, cnts, bsrc2, bdst2, cnts2)[:N]
    xn2, a2, b2 = _tc_comb_set(a1, m1, cvs[1], gas[1], wcs[2])
    m2 = _segmax(pad(b2), bsrc, bdst, cnts, bsrc2, bdst2, cnts2)[:N]
    xn3, a3, b3 = _tc_comb_sub(xn2, a2, m2, cvs[2], gas[2], wcs[3])
    m3 = _segmax(pad(b3), bsrc, bdst, cnts, bsrc2, bdst2, cnts2)[:N]
    return _tc_final(xn3, a3, m3, cvs[3], gas[3], w1, c1)
